# baseline (speedup 1.0000x reference)
import functools
import math

import jax
import jax.numpy as jnp
from jax import lax
from jax.experimental import pallas as pl
from jax.experimental.pallas import tpu as pltpu
from jax.experimental.pallas import tpu_sc as plsc

_EMBED_DIM = 64
_SCALE = math.sqrt(_EMBED_DIM)

_NC = 2
_NS = 16
_NW = _NC * _NS

_CI = 2
_NBUF = 2


def _make_kernel(V_pad, R, S):
  r_per_w = R // _NW
  chunks = r_per_w // _CI
  halves = chunks // _NBUF
  chunk_idx = _CI * S
  rows_per_tile = V_pad // _NS

  mesh = plsc.VectorSubcoreMesh(core_axis_name="c", subcore_axis_name="s",
                                num_cores=_NC, num_subcores=_NS)

  @functools.partial(
      pl.kernel,
      mesh=mesh,
      compiler_params=pltpu.CompilerParams(use_tc_tiling_on_sc=False),
      out_type=jax.ShapeDtypeStruct((R, S, _EMBED_DIM), jnp.float32),
      scratch_types=[
          pltpu.VMEM_SHARED((V_pad, _EMBED_DIM), jnp.float32),
          pltpu.VMEM((rows_per_tile, _EMBED_DIM), jnp.float32),
          pltpu.VMEM((_NBUF, chunk_idx), jnp.int32),
          pltpu.VMEM((_NBUF, chunk_idx, _EMBED_DIM), jnp.float32),
          [pltpu.SemaphoreType.DMA] * _NBUF,
          [pltpu.SemaphoreType.DMA] * _NBUF,
          [pltpu.SemaphoreType.DMA] * _NBUF,
      ],
  )
  def k(lut_hbm, idx_hbm, out_hbm, table_sh, scale_v, idx_v, rows_v,
        sems_i, sems_g, sems_w):
    cid = lax.axis_index("c")
    sid = lax.axis_index("s")
    wid = sid * _NC + cid

    row0 = sid * rows_per_tile
    pltpu.sync_copy(lut_hbm.at[pl.ds(row0, rows_per_tile)], scale_v)

    def scale_row(i, _):
      for j in range(_EMBED_DIM // 16):
        scale_v[i, pl.ds(j * 16, 16)] = scale_v[i, pl.ds(j * 16, 16)] * _SCALE
      return 0

    lax.fori_loop(0, rows_per_tile, scale_row, 0)
    pltpu.sync_copy(scale_v, table_sh.at[pl.ds(row0, rows_per_tile)])
    plsc.subcore_barrier()

    flat_base = wid * r_per_w * S
    out_base = wid * r_per_w

    def idx_copy(g, b):
      return pltpu.make_async_copy(
          idx_hbm.at[pl.ds(flat_base + g * chunk_idx, chunk_idx)],
          idx_v.at[b], sems_i[b])

    def gather_copy(b, a):
      return pltpu.make_async_copy(
          table_sh.at[idx_v.at[b].at[pl.ds(a * S, S)]],
          rows_v.at[b].at[pl.ds(a * S, S)], sems_g[b])

    def out_copy(g, b, a):
      return pltpu.make_async_copy(
          rows_v.at[b].at[pl.ds(a * S, S)],
          out_hbm.at[out_base + g * _CI + a], sems_w[b])

    for b in range(_NBUF):
      idx_copy(b, b).start()

    def body(h, _):
      for b in range(_NBUF):
        g = h * _NBUF + b
        bp = (b - 1) % _NBUF
        idx_copy(g, b).wait()

        @pl.when(h > 0)
        def _():
          for a in range(_CI):
            out_copy(g, b, a).wait()

        for a in range(_CI):
          gather_copy(b, a).start()

        @pl.when(g > 0)
        def _():
          for a in range(_CI):
            gather_copy(bp, a).wait()
          for a in range(_CI):
            out_copy(g - 1, bp, a).start()
          @pl.when(g - 1 + _NBUF < chunks)
          def _():
            idx_copy(g - 1 + _NBUF, bp).start()
      return 0

    lax.fori_loop(0, halves, body, 0)

    bl = (chunks - 1) % _NBUF
    for a in range(_CI):
      gather_copy(bl, a).wait()
    for a in range(_CI):
      out_copy(chunks - 1, bl, a).start()
    for b in range(_NBUF):
      for a in range(_CI):
        out_copy(chunks - _NBUF + b, b, a).wait()

  return k


def kernel(x, lut):
  V, D = lut.shape
  R, S = x.shape
  V_pad = -(-V // (_NS * 8)) * (_NS * 8)
  lut_pad = jnp.pad(lut, ((0, V_pad - V), (0, 0)))
  idx_flat = x.reshape(-1).astype(jnp.int32)
  return _make_kernel(V_pad, R, S)(lut_pad, idx_flat)
